# Initial kernel scaffold; baseline (speedup 1.0000x reference)
import jax
import jax.numpy as jnp
from jax.experimental import pallas as pl


def kernel(node_feature, edge_index, W1, b1, W2, b2):
    raise NotImplementedError("write your pallas kernel here")


# speedup vs baseline: 4.7000x; 4.7000x over previous
import functools

import jax
import jax.numpy as jnp
from jax import lax
from jax.experimental import pallas as pl
from jax.experimental.pallas import tpu as pltpu
from jax.experimental.pallas import tpu_sc as plsc

N_NODES = 10000
N_EDGES = 320000
DIM = 128

NC = 2
NS = 16
NW = NC * NS
CHUNK = 128
N_CHUNKS = N_EDGES // CHUNK
BASE_CH = N_CHUNKS // NW
EXTRA = N_CHUNKS - BASE_CH * NW
N_PAD = 10240
ROWS_PER_TILE = N_PAD // NS
DUMP = 128
DEG_W = 16
IN_COL = 64

_MESH = plsc.VectorSubcoreMesh(core_axis_name="c", subcore_axis_name="s")


def _worker():
    c = lax.axis_index("c")
    s = lax.axis_index("s")
    wid = c * NS + s
    start = wid * BASE_CH + jnp.minimum(wid, EXTRA)
    count = BASE_CH + jnp.where(wid < EXTRA, 1, 0)
    return c, s, start, count


@functools.partial(
    pl.kernel,
    out_type=jax.ShapeDtypeStruct((NC, N_PAD, DIM), jnp.float32),
    mesh=_MESH,
    scratch_types=[
        pltpu.VMEM((2, CHUNK), jnp.int32),
        pltpu.VMEM((CHUNK, DIM), jnp.float32),
        pltpu.VMEM((CHUNK, DIM), jnp.float32),
        pltpu.VMEM_SHARED((N_PAD, DIM), jnp.float32),
    ],
)
def _deg_kernel(src_hbm, dst_hbm, deg_hbm, idx_buf, ones_a, ones_b, acc):
    c, s, start, count = _worker()

    def fill(i, _):
        for k in range(DIM // 16):
            v = 1.0 if k == 0 else 0.0
            ones_a[i, pl.ds(k * 16, 16)] = jnp.full((16,), v, jnp.float32)
            ones_b[i, pl.ds(k * 16, 16)] = jnp.zeros((16,), jnp.float32)
        return 0
    lax.fori_loop(0, CHUNK, fill, 0)

    r0 = s * ROWS_PER_TILE
    for j in range(ROWS_PER_TILE // DUMP):
        pltpu.sync_copy(ones_b, acc.at[pl.ds(r0 + j * DUMP, DUMP)])
    plsc.subcore_barrier()

    def fill_b(i, _):
        ones_b[i, pl.ds(IN_COL, 16)] = jnp.ones((16,), jnp.float32)
        return 0
    lax.fori_loop(0, CHUNK, fill_b, 0)

    def body(j, _):
        @pl.when(j < count)
        def _():
            base = (start + j) * CHUNK
            pltpu.sync_copy(src_hbm.at[pl.ds(base, CHUNK)], idx_buf.at[0])
            pltpu.sync_copy(ones_a, acc.at[idx_buf.at[0]], add=True)
            pltpu.sync_copy(dst_hbm.at[pl.ds(base, CHUNK)], idx_buf.at[1])
            pltpu.sync_copy(ones_b, acc.at[idx_buf.at[1]], add=True)
        return 0
    lax.fori_loop(0, BASE_CH + 1, body, 0)
    plsc.subcore_barrier()

    for j in range(ROWS_PER_TILE // DUMP):
        pltpu.sync_copy(acc.at[pl.ds(r0 + j * DUMP, DUMP)], ones_b)
        pltpu.sync_copy(ones_b, deg_hbm.at[c, pl.ds(r0 + j * DUMP, DUMP)])


@functools.partial(
    pl.kernel,
    out_type=jax.ShapeDtypeStruct((NC, N_PAD, DIM), jnp.float32),
    mesh=_MESH,
    scratch_types=[
        pltpu.VMEM((2, CHUNK), jnp.int32),
        pltpu.VMEM((2, CHUNK), jnp.int32),
        pltpu.VMEM((2, CHUNK, DIM), jnp.float32),
        pltpu.VMEM_SHARED((N_PAD, DIM), jnp.float32),
        pltpu.SemaphoreType.DMA,
    ],
)
def _gs_kernel(y_hbm, src_hbm, dst_hbm, out_hbm,
               idx_s, idx_d, rows, acc, gsem):
    c, s, start, count = _worker()
    stage = rows.at[0]

    def fill_zero(i, _):
        for k in range(DIM // 16):
            rows[0, i, pl.ds(k * 16, 16)] = jnp.zeros((16,), jnp.float32)
        return 0
    lax.fori_loop(0, DUMP, fill_zero, 0)

    r0 = s * ROWS_PER_TILE
    for j in range(ROWS_PER_TILE // DUMP):
        pltpu.sync_copy(stage, acc.at[pl.ds(r0 + j * DUMP, DUMP)])
    plsc.subcore_barrier()

    def body(j, _):
        @pl.when(j < count)
        def _():
            base = (start + j) * CHUNK
            pltpu.sync_copy(src_hbm.at[pl.ds(base, CHUNK)], idx_s.at[0])
            pltpu.async_copy(y_hbm.at[idx_s.at[0]], rows.at[0], gsem).wait()
            pltpu.sync_copy(dst_hbm.at[pl.ds(base, CHUNK)], idx_d.at[0])
            pltpu.sync_copy(rows.at[0], acc.at[idx_d.at[0]], add=True)
        return 0
    lax.fori_loop(0, BASE_CH + 1, body, 0)
    plsc.subcore_barrier()

    for j in range(ROWS_PER_TILE // DUMP):
        pltpu.sync_copy(acc.at[pl.ds(r0 + j * DUMP, DUMP)], stage)
        pltpu.sync_copy(stage, out_hbm.at[c, pl.ds(r0 + j * DUMP, DUMP)])


def _prep_body(deg_ref, x_ref, y_ref, oinv_ref, iinv_ref):
    od = deg_ref[0, :N_NODES, 0:1] + deg_ref[1, :N_NODES, 0:1]
    idg = (deg_ref[0, :N_NODES, IN_COL:IN_COL + 1]
           + deg_ref[1, :N_NODES, IN_COL:IN_COL + 1])
    oinv = lax.rsqrt(jnp.maximum(od, 1.0))
    iinv = lax.rsqrt(jnp.maximum(idg, 1.0))
    oinv_ref[...] = jnp.broadcast_to(oinv, (N_NODES, DEG_W))
    iinv_ref[...] = jnp.broadcast_to(iinv, (N_NODES, DEG_W))
    y_ref[...] = x_ref[...] * oinv


def _prep_call(deg, x):
    return pl.pallas_call(
        _prep_body,
        out_shape=(
            jax.ShapeDtypeStruct((N_NODES, DIM), jnp.float32),
            jax.ShapeDtypeStruct((N_NODES, DEG_W), jnp.float32),
            jax.ShapeDtypeStruct((N_NODES, DEG_W), jnp.float32),
        ),
    )(deg, x)


def _mid_body(acc_ref, iinv_ref, oinv_ref, w_ref, b_ref, y2_ref):
    a = acc_ref[0, :N_NODES] + acc_ref[1, :N_NODES]
    agg = a * iinv_ref[...][:, :1]
    o = jnp.dot(agg, w_ref[...], preferred_element_type=jnp.float32)
    h = jnp.maximum(o + b_ref[...], 0.0)
    y2_ref[...] = h * oinv_ref[...][:, :1]


def _mid_call(acc, iinv, oinv, W, b):
    return pl.pallas_call(
        _mid_body,
        out_shape=jax.ShapeDtypeStruct((N_NODES, DIM), jnp.float32),
    )(acc, iinv, oinv, W, b.reshape(1, DIM))


def _final_body(acc_ref, iinv_ref, w_ref, b_ref, out_ref):
    a = acc_ref[0, :N_NODES] + acc_ref[1, :N_NODES]
    agg = a * iinv_ref[...][:, :1]
    o = jnp.dot(agg, w_ref[...], preferred_element_type=jnp.float32)
    out_ref[...] = jnp.maximum(o + b_ref[...], 0.0)


def _final_call(acc, iinv, W, b):
    return pl.pallas_call(
        _final_body,
        out_shape=jax.ShapeDtypeStruct((N_NODES, DIM), jnp.float32),
    )(acc, iinv, W, b.reshape(1, DIM))


def kernel(node_feature, edge_index, W1, b1, W2, b2):
    ei = edge_index.astype(jnp.int32)
    src = ei[0]
    dst = ei[1]
    deg = _deg_kernel(src, dst)
    y1, oinv, iinv = _prep_call(deg, node_feature)
    acc1 = _gs_kernel(y1, src, dst)
    y2 = _mid_call(acc1, iinv, oinv, W1, b1)
    acc2 = _gs_kernel(y2, src, dst)
    return _final_call(acc2, iinv, W2, b2)
